# Initial kernel scaffold; baseline (speedup 1.0000x reference)
import jax
import jax.numpy as jnp
from jax.experimental import pallas as pl


def kernel(loc_data, conf_data, dbox_list):
    raise NotImplementedError("write your pallas kernel here")


# speedup vs baseline: 27.4440x; 27.4440x over previous
import functools

import jax
import jax.numpy as jnp
from jax import lax
from jax.experimental import pallas as pl
from jax.experimental.pallas import tpu as pltpu
from jax.experimental.pallas import tpu_sc as plsc

CONF_THRESH = 0.01
TOP_K = 200
NMS_THRESH = 0.45
KEEP_TOP_K = 200
BUF = 256
BIG = 1 << 30
ONE_BITS = 0x3F800000
B, P, C = 4, 20000, 21
NCL = C - 1
NVEC = P // 16
OUTW = 21056


def _s(x):
    return x if getattr(x, "ndim", 0) == 0 else x[0]


def _rd(ref, idx):
    base = (idx // 16) * 16
    vec = ref[pl.ds(base, 16)]
    return vec[jnp.full((16,), idx - base, jnp.int32)][0]


def _dense_body(conf_ref, loc_ref, dbox_ref, sbits_ref, boxes_ref,
                vmeta_ref, emeta_ref):
    conf = conf_ref[...][0]
    locv = loc_ref[...][0]
    dbox = dbox_ref[...]
    xy = dbox[0:2] + locv[0:2] * 0.1 * dbox[2:4]
    wh = dbox[2:4] * jnp.exp(locv[2:4] * 0.2)
    x1y1 = xy - wh / 2.0
    x2y2 = x1y1 + wh
    boxes = jnp.clip(jnp.concatenate([x1y1, x2y2], axis=0), 0.0, 1.0)
    boxes_ref[...] = boxes[None]
    m = jnp.max(conf, axis=0, keepdims=True)
    e = jnp.exp(conf - m)
    s = jnp.sum(e, axis=0, keepdims=True)
    probs = e / s
    pc = probs[1:21]
    skey = jnp.where(pc > CONF_THRESH, pc, 0.0)
    bits = lax.bitcast_convert_type(skey, jnp.int32)
    sbits_ref[...] = bits[None]
    lo0 = jnp.full((NCL, 1), -1, jnp.int32)
    hi0 = jnp.full((NCL, 1), ONE_BITS, jnp.int32)

    def bs_body(_, lohi):
        lo, hi = lohi
        mid = (lo + hi) >> 1
        cnt = jnp.sum((bits > mid).astype(jnp.int32), axis=1, keepdims=True)
        small = cnt < TOP_K
        return jnp.where(small, lo, mid), jnp.where(small, mid, hi)

    _, v = lax.fori_loop(0, 32, bs_body, (lo0, hi0))
    n_gt = jnp.sum((bits > v).astype(jnp.int32), axis=1, keepdims=True)
    n_eq = jnp.sum((bits == v).astype(jnp.int32), axis=1, keepdims=True)
    eqt = n_eq - (TOP_K - n_gt)
    eqt = jnp.where(v == 0, BIG, eqt)
    pad = jnp.zeros((24 - NCL, 128), jnp.int32)
    vmeta_ref[...] = jnp.concatenate(
        [jnp.broadcast_to(v, (NCL, 128)), pad], axis=0)[None]
    emeta_ref[...] = jnp.concatenate(
        [jnp.broadcast_to(eqt, (NCL, 128)), pad], axis=0)[None]


def _dense_phase(conf_t, loc_t, dbox_t):
    return pl.pallas_call(
        _dense_body,
        grid=(B,),
        in_specs=[
            pl.BlockSpec((1, C, P), lambda i: (i, 0, 0)),
            pl.BlockSpec((1, 4, P), lambda i: (i, 0, 0)),
            pl.BlockSpec((4, P), lambda i: (0, 0)),
        ],
        out_specs=[
            pl.BlockSpec((1, NCL, P), lambda i: (i, 0, 0)),
            pl.BlockSpec((1, 4, P), lambda i: (i, 0, 0)),
            pl.BlockSpec((1, 24, 128), lambda i: (i, 0, 0)),
            pl.BlockSpec((1, 24, 128), lambda i: (i, 0, 0)),
        ],
        out_shape=[
            jax.ShapeDtypeStruct((B, NCL, P), jnp.int32),
            jax.ShapeDtypeStruct((B, 4, P), jnp.float32),
            jax.ShapeDtypeStruct((B, 24, 128), jnp.int32),
            jax.ShapeDtypeStruct((B, 24, 128), jnp.int32),
        ],
    )(conf_t, loc_t, dbox_t)


def _sc_nms_body(sbits_hbm, vmeta_hbm, emeta_hbm, boxes_hbm,
                 kidx_hbm, kbits_hbm,
                 p0, p1, p2, p3, srow, mrow, erow,
                 cidx, cbits, cx1, cy1, cx2, cy2, car, kiv, kbv):
    wid = lax.axis_index("s") * 2 + lax.axis_index("c")
    img = wid // 8
    k8 = wid % 8
    planes = (p0, p1, p2, p3)
    for coord in range(4):
        pltpu.sync_copy(boxes_hbm.at[img * 4 + coord], planes[coord])
    io16 = lax.iota(jnp.int32, 16)
    zero16 = jnp.zeros((16,), jnp.int32)

    for t in range(3):
        c = k8 + 8 * t
        inst = img * NCL + c

        @pl.when(c < NCL)
        def _instance():
            pltpu.sync_copy(sbits_hbm.at[inst], srow)
            pltpu.sync_copy(vmeta_hbm.at[img * 24 + c], mrow)
            pltpu.sync_copy(emeta_hbm.at[img * 24 + c], erow)
            v = mrow[pl.ds(0, 16)][0]
            eqt = erow[pl.ds(0, 16)][0]
            for j in range(BUF // 16):
                cidx[pl.ds(j * 16, 16)] = zero16
                cbits[pl.ds(j * 16, 16)] = zero16
                kiv[pl.ds(j * 16, 16)] = zero16
                kbv[pl.ds(j * 16, 16)] = zero16

            vs = jnp.full((16,), v, jnp.int32)
            eqts = eqt

            def cbody(j, carry):
                off, eqrun = carry
                vec = srow[pl.ds(j * 16, 16)]
                gt = vec > vs
                eq = vec == vs
                eqc = plsc.cumsum(jnp.where(eq, 1, 0))
                sel = gt | (eq & ((eqc + eqrun) > eqts))
                csel = plsc.cumsum(jnp.where(sel, 1, 0))
                pos = csel - 1 + off
                plsc.store_scatter(cidx, [pos], io16 + j * 16, mask=sel)
                plsc.store_scatter(cbits, [pos], vec, mask=sel)
                return off + csel[15], eqrun + eqc[15]

            lax.fori_loop(0, NVEC, cbody, (jnp.int32(0), jnp.int32(0)))

            for j in range(BUF // 16):
                sl = pl.ds(j * 16, 16)
                iv = cidx[sl]
                x1v = plsc.load_gather(p0, [iv])
                y1v = plsc.load_gather(p1, [iv])
                x2v = plsc.load_gather(p2, [iv])
                y2v = plsc.load_gather(p3, [iv])
                cx1[sl] = x1v
                cy1[sl] = y1v
                cx2[sl] = x2v
                cy2[sl] = y2v
                car[sl] = (x2v - x1v) * (y2v - y1v)

            def wcond(st):
                return st[1]

            def wbody(st):
                kc, _go = st
                mv = cbits[pl.ds(0, 16)]
                for j in range(1, BUF // 16):
                    mv = jnp.maximum(mv, cbits[pl.ds(j * 16, 16)])
                m = jnp.max(mv)

                def do_pick(kc):
                    ms = jnp.full((16,), m, jnp.int32)
                    best = jnp.int32(0)
                    for j in range(BUF // 16):
                        eqm = cbits[pl.ds(j * 16, 16)] == ms
                        mx = jnp.max(jnp.where(eqm, io16, -1))
                        best = jnp.where(mx >= 0, j * 16 + mx, best)
                    g = _rd(cidx, best)
                    lane0 = io16 == 0
                    kcv = jnp.full((16,), kc, jnp.int32)
                    plsc.store_scatter(kiv, [kcv],
                                       jnp.full((16,), g, jnp.int32),
                                       mask=lane0)
                    plsc.store_scatter(kbv, [kcv], ms, mask=lane0)
                    x1i = jnp.full((16,), _rd(cx1, best), jnp.float32)
                    y1i = jnp.full((16,), _rd(cy1, best), jnp.float32)
                    x2i = jnp.full((16,), _rd(cx2, best), jnp.float32)
                    y2i = jnp.full((16,), _rd(cy2, best), jnp.float32)
                    ai = jnp.full((16,), _rd(car, best), jnp.float32)
                    for j in range(BUF // 16):
                        sl = pl.ds(j * 16, 16)
                        xx1 = jnp.maximum(cx1[sl], x1i)
                        yy1 = jnp.maximum(cy1[sl], y1i)
                        xx2 = jnp.minimum(cx2[sl], x2i)
                        yy2 = jnp.minimum(cy2[sl], y2i)
                        w_ = jnp.maximum(xx2 - xx1, 0.0)
                        h_ = jnp.maximum(yy2 - yy1, 0.0)
                        inter = w_ * h_
                        union = car[sl] - inter + ai
                        iou = inter / union
                        keepm = iou <= NMS_THRESH
                        cbits[sl] = jnp.where(keepm, cbits[sl], 0)
                    return kc + 1

                kc2 = lax.cond(m > 0, do_pick, lambda kc: kc, kc)
                return kc2, (m > 0) & (kc2 < TOP_K)

            lax.while_loop(wcond, wbody, (jnp.int32(0), jnp.bool_(True)))
            pltpu.sync_copy(kiv, kidx_hbm.at[inst])
            pltpu.sync_copy(kbv, kbits_hbm.at[inst])


def _sc_nms_phase(sbits, vmeta, emeta, boxesf):
    mesh = plsc.VectorSubcoreMesh(core_axis_name="c", subcore_axis_name="s", num_cores=2, num_subcores=16)
    kern = functools.partial(
        pl.kernel,
        out_type=[
            jax.ShapeDtypeStruct((B * NCL, BUF), jnp.int32),
            jax.ShapeDtypeStruct((B * NCL, BUF), jnp.int32),
        ],
        mesh=mesh,
        compiler_params=pltpu.CompilerParams(needs_layout_passes=False),
        scratch_types=[
            pltpu.VMEM((P,), jnp.float32),
            pltpu.VMEM((P,), jnp.float32),
            pltpu.VMEM((P,), jnp.float32),
            pltpu.VMEM((P,), jnp.float32),
            pltpu.VMEM((P,), jnp.int32),
            pltpu.VMEM((128,), jnp.int32),
            pltpu.VMEM((128,), jnp.int32),
            pltpu.VMEM((BUF,), jnp.int32),
            pltpu.VMEM((BUF,), jnp.int32),
            pltpu.VMEM((BUF,), jnp.float32),
            pltpu.VMEM((BUF,), jnp.float32),
            pltpu.VMEM((BUF,), jnp.float32),
            pltpu.VMEM((BUF,), jnp.float32),
            pltpu.VMEM((BUF,), jnp.float32),
            pltpu.VMEM((BUF,), jnp.int32),
            pltpu.VMEM((BUF,), jnp.int32),
        ],
    )(_sc_nms_body)
    return kern(sbits, vmeta, emeta, boxesf)


def _sc_merge_body(kidx_hbm, kbits_hbm, boxes_hbm, out_hbm,
                   p0, p1, p2, p3, ki, kb, slab, heads, hs):
    wid = lax.axis_index("s") * 2 + lax.axis_index("c")
    io16 = lax.iota(jnp.int32, 16)

    @pl.when(wid < B)
    def _image():
        img = wid
        planes = (p0, p1, p2, p3)
        for coord in range(4):
            pltpu.sync_copy(boxes_hbm.at[img * 4 + coord], planes[coord])
        for c in range(NCL):
            pltpu.sync_copy(kidx_hbm.at[img * NCL + c],
                            ki.at[pl.ds(c * BUF, BUF)])
            pltpu.sync_copy(kbits_hbm.at[img * NCL + c],
                            kb.at[pl.ds(c * BUF, BUF)])

        def zbody(i, _):
            slab[pl.ds(i * 16, 16)] = jnp.zeros((16,), jnp.float32)
            return 0

        lax.fori_loop(0, OUTW // 16, zbody, 0)
        heads[pl.ds(0, 16)] = jnp.zeros((16,), jnp.int32)
        heads[pl.ds(16, 16)] = jnp.zeros((16,), jnp.int32)
        hs[pl.ds(0, 16)] = plsc.load_gather(kb, [io16 * BUF])
        idx1 = jnp.where(io16 < NCL - 16, (io16 + 16) * BUF, 0)
        g1 = plsc.load_gather(kb, [idx1])
        hs[pl.ds(16, 16)] = jnp.where(io16 < NCL - 16, g1, 0)

        def wcond(st):
            return st[1]

        def wbody(st):
            t, _go = st
            v0 = hs[pl.ds(0, 16)]
            v1 = hs[pl.ds(16, 16)]
            m = jnp.max(jnp.maximum(v0, v1))

            def emit(t):
                ms = jnp.full((16,), m, jnp.int32)
                eq0 = v0 == ms
                eq1 = v1 == ms
                n0 = _s(plsc.all_reduce_population_count(eq0))
                c = jnp.where(n0 > 0, _s(plsc.all_reduce_ffs(eq0)),
                              16 + _s(plsc.all_reduce_ffs(eq1)))
                rank = _rd(heads, c)
                g = _rd(ki, c * BUF + rank)
                x1 = jnp.full((16,), _rd(p0, g), jnp.float32)
                y1 = jnp.full((16,), _rd(p1, g), jnp.float32)
                x2 = jnp.full((16,), _rd(p2, g), jnp.float32)
                y2 = jnp.full((16,), _rd(p3, g), jnp.float32)
                sf = plsc.bitcast(ms, jnp.float32)
                vals = jnp.where(io16 == 0, sf,
                       jnp.where(io16 == 1, x1,
                       jnp.where(io16 == 2, y1,
                       jnp.where(io16 == 3, x2, y2))))
                base = ((c + 1) * KEEP_TOP_K + rank) * 5
                plsc.store_scatter(slab, [jnp.full((16,), base, jnp.int32) + io16],
                                   vals, mask=io16 < 5)
                newsc = _rd(kb, c * BUF + rank + 1)
                cs = jnp.full((16,), c, jnp.int32)
                lane0 = io16 == 0
                plsc.store_scatter(heads, [cs],
                                   jnp.full((16,), rank + 1, jnp.int32),
                                   mask=lane0)
                plsc.store_scatter(hs, [cs],
                                   jnp.full((16,), newsc, jnp.int32),
                                   mask=lane0)
                return t + 1

            t2 = lax.cond(m > 0, emit, lambda t: t, t)
            return t2, (m > 0) & (t2 < KEEP_TOP_K)

        lax.while_loop(wcond, wbody, (jnp.int32(0), jnp.bool_(True)))
        pltpu.sync_copy(slab, out_hbm.at[img])


def _sc_merge_phase(kidx, kbits, boxesf):
    mesh = plsc.VectorSubcoreMesh(core_axis_name="c", subcore_axis_name="s", num_cores=2, num_subcores=16)
    kern = functools.partial(
        pl.kernel,
        out_type=jax.ShapeDtypeStruct((B, OUTW), jnp.float32),
        mesh=mesh,
        compiler_params=pltpu.CompilerParams(needs_layout_passes=False),
        scratch_types=[
            pltpu.VMEM((P,), jnp.float32),
            pltpu.VMEM((P,), jnp.float32),
            pltpu.VMEM((P,), jnp.float32),
            pltpu.VMEM((P,), jnp.float32),
            pltpu.VMEM((NCL * BUF,), jnp.int32),
            pltpu.VMEM((NCL * BUF,), jnp.int32),
            pltpu.VMEM((OUTW,), jnp.float32),
            pltpu.VMEM((32,), jnp.int32),
            pltpu.VMEM((32,), jnp.int32),
        ],
    )(_sc_merge_body)
    return kern(kidx, kbits, boxesf)


def kernel(loc_data, conf_data, dbox_list):
    loc = jnp.asarray(loc_data, jnp.float32)
    conf = jnp.asarray(conf_data, jnp.float32)
    dbox = jnp.asarray(dbox_list, jnp.float32)
    conf_t = conf.transpose(0, 2, 1)
    loc_t = loc.transpose(0, 2, 1)
    dbox_t = dbox.T
    sbits, boxes, vmeta, emeta = _dense_phase(conf_t, loc_t, dbox_t)
    kidx, kbits = _sc_nms_phase(
        sbits.reshape(B * NCL, P),
        vmeta.reshape(B * 24, 128),
        emeta.reshape(B * 24, 128),
        boxes.reshape(B * 4, P),
    )
    outflat = _sc_merge_phase(kidx, kbits, boxes.reshape(B * 4, P))
    return outflat[:, : C * KEEP_TOP_K * 5].reshape(B, C, KEEP_TOP_K, 5)


# speedup vs baseline: 29.0744x; 1.0594x over previous
import functools

import jax
import jax.numpy as jnp
from jax import lax
from jax.experimental import pallas as pl
from jax.experimental.pallas import tpu as pltpu
from jax.experimental.pallas import tpu_sc as plsc

CONF_THRESH = 0.01
TOP_K = 200
NMS_THRESH = 0.45
KEEP_TOP_K = 200
BUF = 208
BUFH = 256
BIG = 1 << 30
ONE_BITS = 0x3F800000
LO_BITS = 0x3C23D709
B, P, C = 4, 20000, 21
NCL = C - 1
NVEC = P // 16
OUTW = 21056


def _s(x):
    return x if getattr(x, "ndim", 0) == 0 else x[0]


def _rd(ref, idx):
    base = (idx // 16) * 16
    vec = ref[pl.ds(base, 16)]
    return vec[jnp.full((16,), idx - base, jnp.int32)][0]


def _dense_body(conf_ref, loc_ref, dbox_ref, sbits_ref, boxes_ref,
                vmeta_ref, emeta_ref):
    conf = conf_ref[...][0]
    locv = loc_ref[...][0]
    dbox = dbox_ref[...]
    xy = dbox[0:2] + locv[0:2] * 0.1 * dbox[2:4]
    wh = dbox[2:4] * jnp.exp(locv[2:4] * 0.2)
    x1y1 = xy - wh / 2.0
    x2y2 = x1y1 + wh
    boxes = jnp.clip(jnp.concatenate([x1y1, x2y2], axis=0), 0.0, 1.0)
    boxes_ref[...] = boxes[None]
    m = jnp.max(conf, axis=0, keepdims=True)
    e = jnp.exp(conf - m)
    s = jnp.sum(e, axis=0, keepdims=True)
    probs = e / s
    pc = probs[1:21]
    skey = jnp.where(pc > CONF_THRESH, pc, 0.0)
    bits = lax.bitcast_convert_type(skey, jnp.int32)
    sbits_ref[...] = bits[None]
    lo0 = jnp.full((NCL, 1), LO_BITS, jnp.int32)
    hi0 = jnp.full((NCL, 1), ONE_BITS, jnp.int32)

    def bs_body(_, lohi):
        lo, hi = lohi
        mid = (lo + hi) >> 1
        cnt = jnp.sum((bits > mid).astype(jnp.int32), axis=1, keepdims=True)
        small = cnt < TOP_K
        return jnp.where(small, lo, mid), jnp.where(small, mid, hi)

    _, v = lax.fori_loop(0, 26, bs_body, (lo0, hi0))
    nv = jnp.sum((bits > LO_BITS).astype(jnp.int32), axis=1, keepdims=True)
    v = jnp.where(nv < TOP_K, 0, v)
    n_gt = jnp.sum((bits > v).astype(jnp.int32), axis=1, keepdims=True)
    n_eq = jnp.sum((bits == v).astype(jnp.int32), axis=1, keepdims=True)
    eqt = n_eq - (TOP_K - n_gt)
    eqt = jnp.where(v == 0, BIG, eqt)
    pad = jnp.zeros((24 - NCL, 128), jnp.int32)
    vmeta_ref[...] = jnp.concatenate(
        [jnp.broadcast_to(v, (NCL, 128)), pad], axis=0)[None]
    emeta_ref[...] = jnp.concatenate(
        [jnp.broadcast_to(eqt, (NCL, 128)), pad], axis=0)[None]


def _dense_phase(conf_t, loc_t, dbox_t):
    return pl.pallas_call(
        _dense_body,
        grid=(B,),
        in_specs=[
            pl.BlockSpec((1, C, P), lambda i: (i, 0, 0)),
            pl.BlockSpec((1, 4, P), lambda i: (i, 0, 0)),
            pl.BlockSpec((4, P), lambda i: (0, 0)),
        ],
        out_specs=[
            pl.BlockSpec((1, NCL, P), lambda i: (i, 0, 0)),
            pl.BlockSpec((1, 4, P), lambda i: (i, 0, 0)),
            pl.BlockSpec((1, 24, 128), lambda i: (i, 0, 0)),
            pl.BlockSpec((1, 24, 128), lambda i: (i, 0, 0)),
        ],
        out_shape=[
            jax.ShapeDtypeStruct((B, NCL, P), jnp.int32),
            jax.ShapeDtypeStruct((B, 4, P), jnp.float32),
            jax.ShapeDtypeStruct((B, 24, 128), jnp.int32),
            jax.ShapeDtypeStruct((B, 24, 128), jnp.int32),
        ],
    )(conf_t, loc_t, dbox_t)


def _sc_nms_body(sbits_hbm, vmeta_hbm, emeta_hbm, boxes_hbm,
                 kidx_hbm, kbits_hbm,
                 p0, p1, p2, p3, srow, mrow, erow,
                 cidx, cbits, cx1, cy1, cx2, cy2, car, kiv, kbv):
    wid = lax.axis_index("s") * 2 + lax.axis_index("c")
    img = wid // 8
    k8 = wid % 8
    planes = (p0, p1, p2, p3)
    for coord in range(4):
        pltpu.sync_copy(boxes_hbm.at[img * 4 + coord], planes[coord])
    io16 = lax.iota(jnp.int32, 16)
    zero16 = jnp.zeros((16,), jnp.int32)

    for t in range(3):
        c = k8 + 8 * t
        inst = img * NCL + c

        @pl.when(c < NCL)
        def _instance():
            pltpu.sync_copy(sbits_hbm.at[inst], srow)
            pltpu.sync_copy(vmeta_hbm.at[img * 24 + c], mrow)
            pltpu.sync_copy(emeta_hbm.at[img * 24 + c], erow)
            v = mrow[pl.ds(0, 16)][0]
            eqt = erow[pl.ds(0, 16)][0]
            for j in range(BUF // 16):
                cidx[pl.ds(j * 16, 16)] = zero16
                cbits[pl.ds(j * 16, 16)] = zero16
            for j in range(BUFH // 16):
                kiv[pl.ds(j * 16, 16)] = zero16
                kbv[pl.ds(j * 16, 16)] = zero16

            vs = jnp.full((16,), v, jnp.int32)
            eqts = eqt

            def cbody(j, carry):
                off, eqrun = carry
                vec = srow[pl.ds(j * 16, 16)]
                gt = vec > vs
                eq = vec == vs
                eqc = plsc.cumsum(jnp.where(eq, 1, 0))
                sel = gt | (eq & ((eqc + eqrun) > eqts))
                csel = plsc.cumsum(jnp.where(sel, 1, 0))
                pos = csel - 1 + off
                plsc.store_scatter(cidx, [pos], io16 + j * 16, mask=sel)
                plsc.store_scatter(cbits, [pos], vec, mask=sel)
                return off + csel[15], eqrun + eqc[15]

            lax.fori_loop(0, NVEC, cbody, (jnp.int32(0), jnp.int32(0)))

            for j in range(BUF // 16):
                sl = pl.ds(j * 16, 16)
                iv = cidx[sl]
                x1v = plsc.load_gather(p0, [iv])
                y1v = plsc.load_gather(p1, [iv])
                x2v = plsc.load_gather(p2, [iv])
                y2v = plsc.load_gather(p3, [iv])
                cx1[sl] = x1v
                cy1[sl] = y1v
                cx2[sl] = x2v
                cy2[sl] = y2v
                car[sl] = (x2v - x1v) * (y2v - y1v)

            def wcond(st):
                return st[1]

            def wbody(st):
                kc, _go = st
                curmax = cbits[pl.ds(0, 16)]
                vecidx = jnp.zeros((16,), jnp.int32)
                for j in range(1, BUF // 16):
                    vec = cbits[pl.ds(j * 16, 16)]
                    upd = vec >= curmax
                    curmax = jnp.maximum(curmax, vec)
                    vecidx = jnp.where(upd, j, vecidx)
                m = jnp.max(curmax)

                def do_pick(kc):
                    ms = jnp.full((16,), m, jnp.int32)
                    posv = vecidx * 16 + io16
                    best = jnp.max(jnp.where(curmax == ms, posv, -1))
                    g = _rd(cidx, best)
                    lane0 = io16 == 0
                    kcv = jnp.full((16,), kc, jnp.int32)
                    plsc.store_scatter(kiv, [kcv],
                                       jnp.full((16,), g, jnp.int32),
                                       mask=lane0)
                    plsc.store_scatter(kbv, [kcv], ms, mask=lane0)
                    x1i = jnp.full((16,), _rd(cx1, best), jnp.float32)
                    y1i = jnp.full((16,), _rd(cy1, best), jnp.float32)
                    x2i = jnp.full((16,), _rd(cx2, best), jnp.float32)
                    y2i = jnp.full((16,), _rd(cy2, best), jnp.float32)
                    ai = jnp.full((16,), _rd(car, best), jnp.float32)
                    for j in range(BUF // 16):
                        sl = pl.ds(j * 16, 16)
                        xx1 = jnp.maximum(cx1[sl], x1i)
                        yy1 = jnp.maximum(cy1[sl], y1i)
                        xx2 = jnp.minimum(cx2[sl], x2i)
                        yy2 = jnp.minimum(cy2[sl], y2i)
                        w_ = jnp.maximum(xx2 - xx1, 0.0)
                        h_ = jnp.maximum(yy2 - yy1, 0.0)
                        inter = w_ * h_
                        union = car[sl] - inter + ai
                        iou = inter / union
                        keepm = iou <= NMS_THRESH
                        cbits[sl] = jnp.where(keepm, cbits[sl], 0)
                    return kc + 1

                kc2 = lax.cond(m > 0, do_pick, lambda kc: kc, kc)
                return kc2, (m > 0) & (kc2 < TOP_K)

            lax.while_loop(wcond, wbody, (jnp.int32(0), jnp.bool_(True)))
            pltpu.sync_copy(kiv, kidx_hbm.at[inst])
            pltpu.sync_copy(kbv, kbits_hbm.at[inst])


def _sc_nms_phase(sbits, vmeta, emeta, boxesf):
    mesh = plsc.VectorSubcoreMesh(core_axis_name="c", subcore_axis_name="s", num_cores=2, num_subcores=16)
    kern = functools.partial(
        pl.kernel,
        out_type=[
            jax.ShapeDtypeStruct((B * NCL, BUFH), jnp.int32),
            jax.ShapeDtypeStruct((B * NCL, BUFH), jnp.int32),
        ],
        mesh=mesh,
        compiler_params=pltpu.CompilerParams(needs_layout_passes=False),
        scratch_types=[
            pltpu.VMEM((P,), jnp.float32),
            pltpu.VMEM((P,), jnp.float32),
            pltpu.VMEM((P,), jnp.float32),
            pltpu.VMEM((P,), jnp.float32),
            pltpu.VMEM((P,), jnp.int32),
            pltpu.VMEM((128,), jnp.int32),
            pltpu.VMEM((128,), jnp.int32),
            pltpu.VMEM((BUF,), jnp.int32),
            pltpu.VMEM((BUF,), jnp.int32),
            pltpu.VMEM((BUF,), jnp.float32),
            pltpu.VMEM((BUF,), jnp.float32),
            pltpu.VMEM((BUF,), jnp.float32),
            pltpu.VMEM((BUF,), jnp.float32),
            pltpu.VMEM((BUF,), jnp.float32),
            pltpu.VMEM((BUFH,), jnp.int32),
            pltpu.VMEM((BUFH,), jnp.int32),
        ],
    )(_sc_nms_body)
    return kern(sbits, vmeta, emeta, boxesf)


def _sc_merge_body(kidx_hbm, kbits_hbm, boxes_hbm, out_hbm,
                   p0, p1, p2, p3, ki, kb, slab):
    wid = lax.axis_index("s") * 2 + lax.axis_index("c")
    io16 = lax.iota(jnp.int32, 16)

    @pl.when(wid < B)
    def _image():
        img = wid
        planes = (p0, p1, p2, p3)
        for coord in range(4):
            pltpu.sync_copy(boxes_hbm.at[img * 4 + coord], planes[coord])
        for c in range(NCL):
            pltpu.sync_copy(kidx_hbm.at[img * NCL + c],
                            ki.at[pl.ds(c * BUFH, BUFH)])
            pltpu.sync_copy(kbits_hbm.at[img * NCL + c],
                            kb.at[pl.ds(c * BUFH, BUFH)])

        def zbody(i, _):
            slab[pl.ds(i * 16, 16)] = jnp.zeros((16,), jnp.float32)
            return 0

        lax.fori_loop(0, OUTW // 16, zbody, 0)
        v0i = plsc.load_gather(kb, [io16 * BUFH])
        idx1 = jnp.where(io16 < NCL - 16, (io16 + 16) * BUFH, 0)
        v1i = jnp.where(io16 < NCL - 16, plsc.load_gather(kb, [idx1]), 0)
        zi = jnp.zeros((16,), jnp.int32)

        def wcond(st):
            return st[1]

        def wbody(st):
            t, _go, v0, v1, h0, h1 = st
            m = jnp.max(jnp.maximum(v0, v1))

            def emit(args):
                t, v0, v1, h0, h1 = args
                ms = jnp.full((16,), m, jnp.int32)
                eq0 = v0 == ms
                eq1 = v1 == ms
                n0 = _s(plsc.all_reduce_population_count(eq0))
                c = jnp.where(n0 > 0, _s(plsc.all_reduce_ffs(eq0)),
                              16 + _s(plsc.all_reduce_ffs(eq1)))
                c0 = jnp.where(c < 16, c, c - 16)
                cv = jnp.full((16,), c0, jnp.int32)
                rank = jnp.where(c < 16, h0[cv][0], h1[cv][0])
                newsc = _rd(kb, c * BUFH + rank + 1)
                m0 = io16 == c
                m1 = io16 == (c - 16)
                nsv = jnp.full((16,), newsc, jnp.int32)
                return (t + 1,
                        jnp.where(m0, nsv, v0), jnp.where(m1, nsv, v1),
                        jnp.where(m0, rank + 1, h0),
                        jnp.where(m1, rank + 1, h1))

            t2, v0, v1, h0, h1 = lax.cond(
                m > 0, emit, lambda a: a, (t, v0, v1, h0, h1))
            return t2, (m > 0) & (t2 < KEEP_TOP_K), v0, v1, h0, h1

        st = lax.while_loop(
            wcond, wbody, (jnp.int32(0), jnp.bool_(True), v0i, v1i, zi, zi))
        h0, h1 = st[4], st[5]

        for c in range(NCL):
            tk = h0[c] if c < 16 else h1[c - 16]

            def obody(j, _, c=c, tk=tk):
                sl = pl.ds(c * BUFH + j * 16, 16)
                sl_ids = io16 + j * 16
                mask = sl_ids < tk
                sb = kb[sl]
                gv = ki[sl]
                x1v = plsc.load_gather(p0, [gv])
                y1v = plsc.load_gather(p1, [gv])
                x2v = plsc.load_gather(p2, [gv])
                y2v = plsc.load_gather(p3, [gv])
                base = ((c + 1) * KEEP_TOP_K + sl_ids) * 5
                plsc.store_scatter(slab, [base],
                                   plsc.bitcast(sb, jnp.float32), mask=mask)
                plsc.store_scatter(slab, [base + 1], x1v, mask=mask)
                plsc.store_scatter(slab, [base + 2], y1v, mask=mask)
                plsc.store_scatter(slab, [base + 3], x2v, mask=mask)
                plsc.store_scatter(slab, [base + 4], y2v, mask=mask)
                return 0

            lax.fori_loop(0, BUF // 16, obody, 0)
        pltpu.sync_copy(slab, out_hbm.at[img])


def _sc_merge_phase(kidx, kbits, boxesf):
    mesh = plsc.VectorSubcoreMesh(core_axis_name="c", subcore_axis_name="s", num_cores=2, num_subcores=16)
    kern = functools.partial(
        pl.kernel,
        out_type=jax.ShapeDtypeStruct((B, OUTW), jnp.float32),
        mesh=mesh,
        compiler_params=pltpu.CompilerParams(needs_layout_passes=False),
        scratch_types=[
            pltpu.VMEM((P,), jnp.float32),
            pltpu.VMEM((P,), jnp.float32),
            pltpu.VMEM((P,), jnp.float32),
            pltpu.VMEM((P,), jnp.float32),
            pltpu.VMEM((NCL * BUFH,), jnp.int32),
            pltpu.VMEM((NCL * BUFH,), jnp.int32),
            pltpu.VMEM((OUTW,), jnp.float32),
        ],
    )(_sc_merge_body)
    return kern(kidx, kbits, boxesf)


def kernel(loc_data, conf_data, dbox_list):
    loc = jnp.asarray(loc_data, jnp.float32)
    conf = jnp.asarray(conf_data, jnp.float32)
    dbox = jnp.asarray(dbox_list, jnp.float32)
    conf_t = conf.transpose(0, 2, 1)
    loc_t = loc.transpose(0, 2, 1)
    dbox_t = dbox.T
    sbits, boxes, vmeta, emeta = _dense_phase(conf_t, loc_t, dbox_t)
    kidx, kbits = _sc_nms_phase(
        sbits.reshape(B * NCL, P),
        vmeta.reshape(B * 24, 128),
        emeta.reshape(B * 24, 128),
        boxes.reshape(B * 4, P),
    )
    outflat = _sc_merge_phase(kidx, kbits, boxes.reshape(B * 4, P))
    return outflat[:, : C * KEEP_TOP_K * 5].reshape(B, C, KEEP_TOP_K, 5)


# speedup vs baseline: 36.3102x; 1.2489x over previous
import functools

import jax
import jax.numpy as jnp
from jax import lax
from jax.experimental import pallas as pl
from jax.experimental.pallas import tpu as pltpu
from jax.experimental.pallas import tpu_sc as plsc

CONF_THRESH = 0.01
TOP_K = 200
NMS_THRESH = 0.45
KEEP_TOP_K = 200
BUF = 208
BUFPAD = 224
BUFH = 256
BIG = 1 << 30
ONE_BITS = 0x3F800000
LO_BITS = 0x3C23D709
B, P, C = 4, 20000, 21
NCL = C - 1
NVEC = P // 16
OUTW = 21056


def _s(x):
    return x if getattr(x, "ndim", 0) == 0 else x[0]


def _rd(ref, idx):
    base = (idx // 16) * 16
    vec = ref[pl.ds(base, 16)]
    return vec[jnp.full((16,), idx - base, jnp.int32)][0]


def _dense_body(conf_ref, loc_ref, dbox_ref, sbits_ref, boxes_ref,
                vmeta_ref, emeta_ref):
    conf = conf_ref[...][0]
    locv = loc_ref[...][0]
    dbox = dbox_ref[...]
    xy = dbox[0:2] + locv[0:2] * 0.1 * dbox[2:4]
    wh = dbox[2:4] * jnp.exp(locv[2:4] * 0.2)
    x1y1 = xy - wh / 2.0
    x2y2 = x1y1 + wh
    boxes = jnp.clip(jnp.concatenate([x1y1, x2y2], axis=0), 0.0, 1.0)
    boxes_ref[...] = boxes[None]
    m = jnp.max(conf, axis=0, keepdims=True)
    e = jnp.exp(conf - m)
    s = jnp.sum(e, axis=0, keepdims=True)
    probs = e / s
    pc = probs[1:21]
    skey = jnp.where(pc > CONF_THRESH, pc, 0.0)
    bits = lax.bitcast_convert_type(skey, jnp.int32)
    sbits_ref[...] = bits[None]
    lo0 = jnp.full((NCL, 1), LO_BITS, jnp.int32)
    hi0 = jnp.full((NCL, 1), ONE_BITS, jnp.int32)

    def bs_body(_, lohi):
        lo, hi = lohi
        mid = (lo + hi) >> 1
        cnt = jnp.sum((bits > mid).astype(jnp.int32), axis=1, keepdims=True)
        small = cnt < TOP_K
        return jnp.where(small, lo, mid), jnp.where(small, mid, hi)

    _, v = lax.fori_loop(0, 26, bs_body, (lo0, hi0))
    nv = jnp.sum((bits > LO_BITS).astype(jnp.int32), axis=1, keepdims=True)
    v = jnp.where(nv < TOP_K, 0, v)
    n_gt = jnp.sum((bits > v).astype(jnp.int32), axis=1, keepdims=True)
    n_eq = jnp.sum((bits == v).astype(jnp.int32), axis=1, keepdims=True)
    eqt = n_eq - (TOP_K - n_gt)
    eqt = jnp.where(v == 0, BIG, eqt)
    pad = jnp.zeros((24 - NCL, 128), jnp.int32)
    vmeta_ref[...] = jnp.concatenate(
        [jnp.broadcast_to(v, (NCL, 128)), pad], axis=0)[None]
    emeta_ref[...] = jnp.concatenate(
        [jnp.broadcast_to(eqt, (NCL, 128)), pad], axis=0)[None]


def _dense_phase(conf_t, loc_t, dbox_t):
    return pl.pallas_call(
        _dense_body,
        grid=(B,),
        in_specs=[
            pl.BlockSpec((1, C, P), lambda i: (i, 0, 0)),
            pl.BlockSpec((1, 4, P), lambda i: (i, 0, 0)),
            pl.BlockSpec((4, P), lambda i: (0, 0)),
        ],
        out_specs=[
            pl.BlockSpec((1, NCL, P), lambda i: (i, 0, 0)),
            pl.BlockSpec((1, 4, P), lambda i: (i, 0, 0)),
            pl.BlockSpec((1, 24, 128), lambda i: (i, 0, 0)),
            pl.BlockSpec((1, 24, 128), lambda i: (i, 0, 0)),
        ],
        out_shape=[
            jax.ShapeDtypeStruct((B, NCL, P), jnp.int32),
            jax.ShapeDtypeStruct((B, 4, P), jnp.float32),
            jax.ShapeDtypeStruct((B, 24, 128), jnp.int32),
            jax.ShapeDtypeStruct((B, 24, 128), jnp.int32),
        ],
    )(conf_t, loc_t, dbox_t)


def _sc_nms_body(sbits_hbm, vmeta_hbm, emeta_hbm, boxes_hbm,
                 kbits_hbm, kx1_hbm, ky1_hbm, kx2_hbm, ky2_hbm,
                 p0, p1, p2, p3, srow, mrow, erow,
                 cidx, cbits, cx1, cy1, cx2, cy2, car,
                 kbv, kx1v, ky1v, kx2v, ky2v):
    wid = lax.axis_index("s") * 2 + lax.axis_index("c")
    img = wid // 8
    k8 = wid % 8
    planes = (p0, p1, p2, p3)
    for coord in range(4):
        pltpu.sync_copy(boxes_hbm.at[img * 4 + coord], planes[coord])
    io16 = lax.iota(jnp.int32, 16)
    zero16 = jnp.zeros((16,), jnp.int32)
    zf16 = jnp.zeros((16,), jnp.float32)

    for t in range(3):
        c = k8 + 8 * t
        inst = img * NCL + c

        @pl.when(c < NCL)
        def _instance():
            pltpu.sync_copy(sbits_hbm.at[inst], srow)
            pltpu.sync_copy(vmeta_hbm.at[img * 24 + c], mrow)
            pltpu.sync_copy(emeta_hbm.at[img * 24 + c], erow)
            v = mrow[pl.ds(0, 16)][0]
            eqt = erow[pl.ds(0, 16)][0]
            for j in range(BUFPAD // 16):
                cidx[pl.ds(j * 16, 16)] = zero16
                cbits[pl.ds(j * 16, 16)] = zero16
            for j in range(BUFH // 16):
                kbv[pl.ds(j * 16, 16)] = zero16
                kx1v[pl.ds(j * 16, 16)] = zf16
                ky1v[pl.ds(j * 16, 16)] = zf16
                kx2v[pl.ds(j * 16, 16)] = zf16
                ky2v[pl.ds(j * 16, 16)] = zf16

            vs = jnp.full((16,), v, jnp.int32)
            allow_eq = jnp.full((16,), v > 0)

            def cbody_fast(j, off):
                vec = srow[pl.ds(j * 16, 16)]
                sel = (vec > vs) | ((vec == vs) & allow_eq)
                plsc.store_compressed(cidx.at[pl.ds(off, 16)],
                                      io16 + j * 16, mask=sel)
                plsc.store_compressed(cbits.at[pl.ds(off, 16)], vec,
                                      mask=sel)
                return off + _s(plsc.all_reduce_population_count(sel))

            def cbody_slow(j, carry):
                off, eqrun = carry
                vec = srow[pl.ds(j * 16, 16)]
                gt = vec > vs
                eq = vec == vs
                eqc = plsc.cumsum(jnp.where(eq, 1, 0))
                sel = gt | (eq & ((eqc + eqrun) > eqt))
                csel = plsc.cumsum(jnp.where(sel, 1, 0))
                pos = csel - 1 + off
                plsc.store_scatter(cidx, [pos], io16 + j * 16, mask=sel)
                plsc.store_scatter(cbits, [pos], vec, mask=sel)
                return off + csel[15], eqrun + eqc[15]

            def fast_path(_):
                return lax.fori_loop(0, NVEC, cbody_fast, jnp.int32(0))

            def slow_path(_):
                o, _e = lax.fori_loop(0, NVEC, cbody_slow,
                                      (jnp.int32(0), jnp.int32(0)))
                return o

            off = lax.cond((v > 0) & (eqt > 0), slow_path, fast_path, 0)
            offv = jnp.full((16,), off, jnp.int32) + io16
            plsc.store_scatter(cidx, [offv], zero16)
            plsc.store_scatter(cbits, [offv], zero16)

            for j in range(BUF // 16):
                sl = pl.ds(j * 16, 16)
                iv = cidx[sl]
                x1v = plsc.load_gather(p0, [iv])
                y1v = plsc.load_gather(p1, [iv])
                x2v = plsc.load_gather(p2, [iv])
                y2v = plsc.load_gather(p3, [iv])
                cx1[sl] = x1v
                cy1[sl] = y1v
                cx2[sl] = x2v
                cy2[sl] = y2v
                car[sl] = (x2v - x1v) * (y2v - y1v)

            def wcond(st):
                return st[1]

            def wbody(st):
                kc, _go = st
                curmax = cbits[pl.ds(0, 16)]
                vecidx = jnp.zeros((16,), jnp.int32)
                for j in range(1, BUF // 16):
                    vec = cbits[pl.ds(j * 16, 16)]
                    upd = vec >= curmax
                    curmax = jnp.maximum(curmax, vec)
                    vecidx = jnp.where(upd, j, vecidx)
                m = jnp.max(curmax)

                def do_pick(kc):
                    ms = jnp.full((16,), m, jnp.int32)
                    posv = vecidx * 16 + io16
                    best = jnp.max(jnp.where(curmax == ms, posv, -1))
                    lane0 = io16 == 0
                    kcv = jnp.full((16,), kc, jnp.int32)
                    x1i = jnp.full((16,), _rd(cx1, best), jnp.float32)
                    y1i = jnp.full((16,), _rd(cy1, best), jnp.float32)
                    x2i = jnp.full((16,), _rd(cx2, best), jnp.float32)
                    y2i = jnp.full((16,), _rd(cy2, best), jnp.float32)
                    ai = jnp.full((16,), _rd(car, best), jnp.float32)
                    plsc.store_scatter(kbv, [kcv], ms, mask=lane0)
                    plsc.store_scatter(kx1v, [kcv], x1i, mask=lane0)
                    plsc.store_scatter(ky1v, [kcv], y1i, mask=lane0)
                    plsc.store_scatter(kx2v, [kcv], x2i, mask=lane0)
                    plsc.store_scatter(ky2v, [kcv], y2i, mask=lane0)
                    for j in range(BUF // 16):
                        sl = pl.ds(j * 16, 16)
                        xx1 = jnp.maximum(cx1[sl], x1i)
                        yy1 = jnp.maximum(cy1[sl], y1i)
                        xx2 = jnp.minimum(cx2[sl], x2i)
                        yy2 = jnp.minimum(cy2[sl], y2i)
                        w_ = jnp.maximum(xx2 - xx1, 0.0)
                        h_ = jnp.maximum(yy2 - yy1, 0.0)
                        inter = w_ * h_
                        union = car[sl] - inter + ai
                        iou = inter / union
                        keepm = iou <= NMS_THRESH
                        cbits[sl] = jnp.where(keepm, cbits[sl], 0)
                    return kc + 1

                kc2 = lax.cond(m > 0, do_pick, lambda kc: kc, kc)
                return kc2, (m > 0) & (kc2 < TOP_K)

            lax.while_loop(wcond, wbody, (jnp.int32(0), jnp.bool_(True)))
            orow = img * 24 + c
            pltpu.sync_copy(kbv, kbits_hbm.at[orow])
            pltpu.sync_copy(kx1v, kx1_hbm.at[orow])
            pltpu.sync_copy(ky1v, ky1_hbm.at[orow])
            pltpu.sync_copy(kx2v, kx2_hbm.at[orow])
            pltpu.sync_copy(ky2v, ky2_hbm.at[orow])


def _sc_nms_phase(sbits, vmeta, emeta, boxesf):
    mesh = plsc.VectorSubcoreMesh(core_axis_name="c", subcore_axis_name="s",
                                  num_cores=2, num_subcores=16)
    f32 = jnp.float32
    kern = functools.partial(
        pl.kernel,
        out_type=[
            jax.ShapeDtypeStruct((B * 24, BUFH), jnp.int32),
            jax.ShapeDtypeStruct((B * 24, BUFH), f32),
            jax.ShapeDtypeStruct((B * 24, BUFH), f32),
            jax.ShapeDtypeStruct((B * 24, BUFH), f32),
            jax.ShapeDtypeStruct((B * 24, BUFH), f32),
        ],
        mesh=mesh,
        compiler_params=pltpu.CompilerParams(needs_layout_passes=False),
        scratch_types=[
            pltpu.VMEM((P,), f32),
            pltpu.VMEM((P,), f32),
            pltpu.VMEM((P,), f32),
            pltpu.VMEM((P,), f32),
            pltpu.VMEM((P,), jnp.int32),
            pltpu.VMEM((128,), jnp.int32),
            pltpu.VMEM((128,), jnp.int32),
            pltpu.VMEM((BUFPAD,), jnp.int32),
            pltpu.VMEM((BUFPAD,), jnp.int32),
            pltpu.VMEM((BUF,), f32),
            pltpu.VMEM((BUF,), f32),
            pltpu.VMEM((BUF,), f32),
            pltpu.VMEM((BUF,), f32),
            pltpu.VMEM((BUF,), f32),
            pltpu.VMEM((BUFH,), jnp.int32),
            pltpu.VMEM((BUFH,), f32),
            pltpu.VMEM((BUFH,), f32),
            pltpu.VMEM((BUFH,), f32),
            pltpu.VMEM((BUFH,), f32),
        ],
    )(_sc_nms_body)
    return kern(sbits, vmeta, emeta, boxesf)


def _sc_merge_body(kbits_hbm, kx1_hbm, ky1_hbm, kx2_hbm, ky2_hbm, out_hbm,
                   kb2, bx1, by1, bx2, by2, slab):
    wid = lax.axis_index("s") * 2 + lax.axis_index("c")
    io16 = lax.iota(jnp.int32, 16)
    zero16 = jnp.zeros((16,), jnp.int32)
    zf16 = jnp.zeros((16,), jnp.float32)

    @pl.when(wid < B)
    def _image():
        img = wid
        rows = pl.ds(img * 24, 24)
        pltpu.sync_copy(kbits_hbm.at[rows], kb2)
        pltpu.sync_copy(kx1_hbm.at[rows], bx1)
        pltpu.sync_copy(ky1_hbm.at[rows], by1)
        pltpu.sync_copy(kx2_hbm.at[rows], bx2)
        pltpu.sync_copy(ky2_hbm.at[rows], by2)

        v0i = plsc.load_gather(kb2, [io16, zero16])
        rows1 = jnp.where(io16 < NCL - 16, io16 + 16, 0)
        v1i = jnp.where(io16 < NCL - 16,
                        plsc.load_gather(kb2, [rows1, zero16]), 0)
        zi = jnp.zeros((16,), jnp.int32)

        def wcond(st):
            return st[1]

        def wbody(st):
            t, _go, v0, v1, h0, h1 = st
            m = jnp.max(jnp.maximum(v0, v1))

            def emit(args):
                t, v0, v1, h0, h1 = args
                ms = jnp.full((16,), m, jnp.int32)
                eq0 = v0 == ms
                eq1 = v1 == ms
                n0 = _s(plsc.all_reduce_population_count(eq0))
                c = jnp.where(n0 > 0, _s(plsc.all_reduce_ffs(eq0)),
                              16 + _s(plsc.all_reduce_ffs(eq1)))
                c0 = jnp.where(c < 16, c, c - 16)
                cv = jnp.full((16,), c0, jnp.int32)
                rank = jnp.where(c < 16, h0[cv][0], h1[cv][0])
                newsc = plsc.load_gather(
                    kb2, [jnp.full((16,), c, jnp.int32),
                          jnp.full((16,), rank + 1, jnp.int32)])[0]
                m0 = io16 == c
                m1 = io16 == (c - 16)
                nsv = jnp.full((16,), newsc, jnp.int32)
                return (t + 1,
                        jnp.where(m0, nsv, v0), jnp.where(m1, nsv, v1),
                        jnp.where(m0, rank + 1, h0),
                        jnp.where(m1, rank + 1, h1))

            t2, v0, v1, h0, h1 = lax.cond(
                m > 0, emit, lambda a: a, (t, v0, v1, h0, h1))
            return t2, (m > 0) & (t2 < KEEP_TOP_K), v0, v1, h0, h1

        st = lax.while_loop(
            wcond, wbody, (jnp.int32(0), jnp.bool_(True), v0i, v1i, zi, zi))
        h0, h1 = st[4], st[5]

        def zrow(j, _):
            sl_ids = io16 + j * 16
            wmask = sl_ids < KEEP_TOP_K
            base = sl_ids * 5
            for fo in range(5):
                plsc.store_scatter(slab, [base + fo], zf16, mask=wmask)
            return 0

        lax.fori_loop(0, BUF // 16, zrow, 0)

        for c in range(NCL):
            tk = h0[c] if c < 16 else h1[c - 16]
            cv = jnp.full((16,), c, jnp.int32)

            def obody(j, _, c=c, tk=tk, cv=cv):
                sl_ids = io16 + j * 16
                wmask = sl_ids < KEEP_TOP_K
                vmask = sl_ids < tk
                sb = plsc.load_gather(kb2, [cv, sl_ids])
                x1v = plsc.load_gather(bx1, [cv, sl_ids])
                y1v = plsc.load_gather(by1, [cv, sl_ids])
                x2v = plsc.load_gather(bx2, [cv, sl_ids])
                y2v = plsc.load_gather(by2, [cv, sl_ids])
                base = ((c + 1) * KEEP_TOP_K + sl_ids) * 5
                zf = jnp.float32(0)
                plsc.store_scatter(
                    slab, [base],
                    jnp.where(vmask, plsc.bitcast(sb, jnp.float32), zf),
                    mask=wmask)
                plsc.store_scatter(slab, [base + 1],
                                   jnp.where(vmask, x1v, zf), mask=wmask)
                plsc.store_scatter(slab, [base + 2],
                                   jnp.where(vmask, y1v, zf), mask=wmask)
                plsc.store_scatter(slab, [base + 3],
                                   jnp.where(vmask, x2v, zf), mask=wmask)
                plsc.store_scatter(slab, [base + 4],
                                   jnp.where(vmask, y2v, zf), mask=wmask)
                return 0

            lax.fori_loop(0, BUF // 16, obody, 0)

        for k in range(4):
            idxp = C * KEEP_TOP_K * 5 + io16 + 16 * k
            plsc.store_scatter(slab, [idxp], zf16, mask=idxp < OUTW)
        pltpu.sync_copy(slab, out_hbm.at[img])


def _sc_merge_phase(kbits, kx1, ky1, kx2, ky2):
    mesh = plsc.VectorSubcoreMesh(core_axis_name="c", subcore_axis_name="s",
                                  num_cores=2, num_subcores=16)
    f32 = jnp.float32
    kern = functools.partial(
        pl.kernel,
        out_type=jax.ShapeDtypeStruct((B, OUTW), f32),
        mesh=mesh,
        compiler_params=pltpu.CompilerParams(needs_layout_passes=False),
        scratch_types=[
            pltpu.VMEM((24, BUFH), jnp.int32),
            pltpu.VMEM((24, BUFH), f32),
            pltpu.VMEM((24, BUFH), f32),
            pltpu.VMEM((24, BUFH), f32),
            pltpu.VMEM((24, BUFH), f32),
            pltpu.VMEM((OUTW,), f32),
        ],
    )(_sc_merge_body)
    return kern(kbits, kx1, ky1, kx2, ky2)


def kernel(loc_data, conf_data, dbox_list):
    loc = jnp.asarray(loc_data, jnp.float32)
    conf = jnp.asarray(conf_data, jnp.float32)
    dbox = jnp.asarray(dbox_list, jnp.float32)
    conf_t = conf.transpose(0, 2, 1)
    loc_t = loc.transpose(0, 2, 1)
    dbox_t = dbox.T
    sbits, boxes, vmeta, emeta = _dense_phase(conf_t, loc_t, dbox_t)
    kbits, kx1, ky1, kx2, ky2 = _sc_nms_phase(
        sbits.reshape(B * NCL, P),
        vmeta.reshape(B * 24, 128),
        emeta.reshape(B * 24, 128),
        boxes.reshape(B * 4, P),
    )
    outflat = _sc_merge_phase(kbits, kx1, ky1, kx2, ky2)
    return outflat[:, : C * KEEP_TOP_K * 5].reshape(B, C, KEEP_TOP_K, 5)
